# baseline (speedup 1.0000x reference)
import functools

import jax
import jax.numpy as jnp
from jax import lax
from jax.experimental import pallas as pl
from jax.experimental.pallas import tpu as pltpu
from jax.experimental.pallas import tpu_sc as plsc


def _silu(v):
    return v * jax.nn.sigmoid(v)


def _ln(v, w, b):
    m = v.mean(-1, keepdims=True)
    var = ((v - m) ** 2).mean(-1, keepdims=True)
    return (v - m) * jax.lax.rsqrt(var + 1e-5) * w + b


def _inproj_body(x_ref, w_ref, lnw_ref, lnb_ref, o_ref):
    xn = _ln(x_ref[...], lnw_ref[...], lnb_ref[...])
    o_ref[...] = jnp.dot(xn, w_ref[...], preferred_element_type=jnp.float32)


def _conv_body(xin_ref, convw_ref, convb_ref, xpw_ref, dtw_ref, dtb_ref,
               xs_ref, dt_ref, bp_ref, cp_ref, carry_ref):
    nb = pl.program_id(0)

    @pl.when(nb == 0)
    def _():
        carry_ref[...] = jnp.zeros_like(carry_ref)

    xin = xin_ref[...]
    ln = xin.shape[0]
    dc = convw_ref.shape[0]
    ext = jnp.concatenate([carry_ref[...], xin], axis=0)
    acc = jnp.broadcast_to(convb_ref[...], xin.shape)
    for k in range(dc):
        acc = acc + convw_ref[k, :][None, :] * ext[8 + k - (dc - 1): 8 + k - (dc - 1) + ln, :]
    xs = _silu(acc)
    xs_ref[...] = xs
    carry_ref[...] = xin[ln - 8: ln, :]
    xp = jnp.dot(xs, xpw_ref[...], preferred_element_type=jnp.float32)
    r = dtw_ref.shape[0]
    s = bp_ref.shape[1]
    bp_ref[...] = xp[:, r: r + s]
    cp_ref[...] = xp[:, r + s: r + 2 * s]
    dt_ref[...] = jax.nn.softplus(
        jnp.dot(xp[:, :r], dtw_ref[...], preferred_element_type=jnp.float32)
        + dtb_ref[...])


def _scan_body(xs_ref, dt_ref, bp_ref, cp_ref, alogt_ref, dp_ref, y_ref,
               h_ref, da_ref, dbx_ref):
    g = pl.program_id(0)

    @pl.when(g == 0)
    def _():
        h_ref[...] = jnp.zeros_like(h_ref)

    a_neg = -jnp.exp(alogt_ref[...])
    dp = dp_ref[...]
    lc = xs_ref.shape[0]
    u = 32
    outer_dn = (((0,), (0,)), ((), ()))
    contr_dn = (((1,), (0,)), ((), ()))

    def body(k, h):
        j0 = pl.multiple_of(k * u, u)
        dt_c = dt_ref[pl.ds(j0, u), :]
        xs_c = xs_ref[pl.ds(j0, u), :]
        bp_c = bp_ref[pl.ds(j0, u), :]
        cp_c = cp_ref[pl.ds(j0, u), :]
        dtx_c = dt_c * xs_c
        for j in range(u):
            da_ref[j] = jnp.exp(dt_c[j:j + 1, :] * a_neg)
            dbx_ref[j] = jax.lax.dot_general(
                bp_c[j:j + 1, :], dtx_c[j:j + 1, :], outer_dn,
                preferred_element_type=jnp.float32)
        rows = []
        for j in range(u):
            h = da_ref[j] * h + dbx_ref[j]
            rows.append(jax.lax.dot_general(cp_c[j:j + 1, :], h, contr_dn,
                                            preferred_element_type=jnp.float32))
        y_ref[pl.ds(j0, u), :] = jnp.concatenate(rows, axis=0) + dp * xs_c
        return h

    h_ref[...] = jax.lax.fori_loop(0, lc // u, body, h_ref[...])


def _post_body(ys_ref, z_ref, opw_ref, x_ref, ln2w_ref, ln2b_ref, rw_ref,
               h_ref, hn_ref, comb_ref, ints_ref, w12_ref):
    y = ys_ref[...] * _silu(z_ref[...])
    h = x_ref[...] + jnp.dot(y, opw_ref[...], preferred_element_type=jnp.float32)
    h_ref[...] = h
    hn = _ln(h, ln2w_ref[...], ln2b_ref[...])
    hn_ref[...] = hn
    logits = jnp.dot(hn, rw_ref[...], preferred_element_type=jnp.float32)
    p = jax.nn.softmax(logits, axis=-1)
    col = jax.lax.broadcasted_iota(jnp.int32, p.shape, 1)
    i1 = jnp.argmax(p, axis=-1, keepdims=True)
    m1 = jnp.max(p, axis=-1, keepdims=True)
    pm = jnp.where(col == i1, -jnp.inf, p)
    i2 = jnp.argmax(pm, axis=-1, keepdims=True)
    m2 = jnp.max(pm, axis=-1, keepdims=True)
    tot = m1 + m2
    comb_ref[...] = jnp.where(col == i1, m1 / tot,
                              jnp.where(col == i2, m2 / tot, 0.0))
    ints_ref[...] = jnp.concatenate([i1, i2], axis=1).astype(jnp.int32)
    w12_ref[...] = jnp.concatenate([m1 / tot, m2 / tot], axis=1)


_RBLK = 128


def _route_body(comb_ref, ints_ref, dest_ref, beq_ref):
    n, e = comb_ref.shape
    nbk = n // _RBLK
    f32 = jnp.float32
    m = (comb_ref[...] != 0.0).astype(f32)
    ri = lax.broadcasted_iota(jnp.int32, (_RBLK, _RBLK), 0)
    ci = lax.broadcasted_iota(jnp.int32, (_RBLK, _RBLK), 1)
    sl = (ci < ri).astype(f32)
    ones_row = jnp.ones((1, _RBLK), f32)
    l16 = (lax.broadcasted_iota(jnp.int32, (nbk, nbk), 1)
           <= lax.broadcasted_iota(jnp.int32, (nbk, nbk), 0)).astype(f32)
    su8 = (lax.broadcasted_iota(jnp.int32, (e, e), 0)
           < lax.broadcasted_iota(jnp.int32, (e, e), 1)).astype(f32)
    dnum = (((1,), (0,)), ((), ()))
    ranks = []
    cnts = []
    for b in range(nbk):
        mb = m[b * _RBLK:(b + 1) * _RBLK, :]
        ranks.append(lax.dot_general(sl, mb, dnum, preferred_element_type=f32))
        cnts.append(lax.dot_general(ones_row, mb, dnum, preferred_element_type=f32))
    pbc = jnp.concatenate(cnts, axis=0)
    p_incl = lax.dot_general(l16, pbc, dnum, preferred_element_type=f32)
    tot = p_incl[nbk - 1:nbk, :]
    ti = tot.astype(jnp.int32)
    ptf = (((ti + (_RBLK - 1)) >> 7) << 7).astype(f32)
    pbase = lax.dot_general(ptf, su8, dnum, preferred_element_type=f32)
    bot = pbase + p_incl - pbc
    col8 = lax.broadcasted_iota(jnp.int32, (_RBLK, e), 1)
    rows = []
    for b in range(nbk):
        dest_b = ranks[b] + bot[b:b + 1, :]
        i1 = ints_ref[b * _RBLK:(b + 1) * _RBLK, 0:1]
        i2 = ints_ref[b * _RBLK:(b + 1) * _RBLK, 1:2]
        d1 = jnp.sum(jnp.where(col8 == i1, dest_b, 0.0), axis=1, keepdims=True)
        d2 = jnp.sum(jnp.where(col8 == i2, dest_b, 0.0), axis=1, keepdims=True)
        rows.append(jnp.concatenate([d1, d2], axis=1))
    dest_ref[...] = jnp.concatenate(rows, axis=0).astype(jnp.int32)
    qlo = lax.broadcasted_iota(jnp.int32, beq_ref.shape, 0).astype(f32) * _RBLK
    ecol = lax.broadcasted_iota(jnp.int32, beq_ref.shape, 1).astype(f32)
    inseg = (qlo >= pbase) & (qlo < pbase + ptf)
    be_col = jnp.sum(jnp.where(inseg, ecol, 0.0), axis=1, keepdims=True)
    beq_ref[...] = jnp.broadcast_to(be_col, beq_ref.shape).astype(jnp.int32)


def _gmm_body(be_ref, xr_ref, wg_ref, wu_ref, wd_ref, o_ref):
    hb = pl.program_id(1)

    @pl.when(hb == 0)
    def _():
        o_ref[...] = jnp.zeros_like(o_ref)

    xb = xr_ref[...].astype(jnp.bfloat16)
    dn = (((1,), (1,)), ((), ()))
    wg = wg_ref[0].astype(jnp.bfloat16)
    wu = wu_ref[0].astype(jnp.bfloat16)
    wd = wd_ref[0].astype(jnp.bfloat16)
    g = lax.dot_general(xb, wg, dn, preferred_element_type=jnp.float32)
    u = lax.dot_general(xb, wu, dn, preferred_element_type=jnp.float32)
    act = (_silu(g) * u).astype(jnp.bfloat16)
    o_ref[...] += lax.dot_general(act, wd, dn, preferred_element_type=jnp.float32)


def _combine_body(eo_ref, h_ref, dest_ref, w_ref, o_ref):
    blk = pl.program_id(0)
    bn = h_ref.shape[0]
    n = bn * pl.num_programs(0)

    def body(j, carry):
        tok = blk * bn + j
        d1 = dest_ref[tok]
        d2 = dest_ref[n + tok]
        w1 = w_ref[tok]
        w2 = w_ref[n + tok]
        o_ref[pl.ds(j, 1), :] = (h_ref[pl.ds(j, 1), :]
                                 + w1 * eo_ref[pl.ds(d1, 1), :]
                                 + w2 * eo_ref[pl.ds(d2, 1), :])
        return carry

    lax.fori_loop(0, bn, body, 0)


def kernel(x, ln1_w, ln1_b, ln2_w, ln2_b, in_proj_w, conv_w, conv_b, x_proj_w,
           dt_proj_w, dt_proj_b, A_log, Dp, out_proj_w, router_w, w_gate, w_up, w_down):
    b, t, d = x.shape
    di = conv_w.shape[0]
    r = dt_proj_w.shape[1]
    s = A_log.shape[1]
    e = router_w.shape[0]
    h_dim = w_gate.shape[1]
    n = b * t
    f32 = jnp.float32
    xf = x.reshape(n, d)

    ln = 256
    cb = 768
    n_nb = n // ln

    xz = pl.pallas_call(
        _inproj_body,
        grid=(n_nb, (2 * di) // cb),
        in_specs=[
            pl.BlockSpec((ln, d), lambda i, j: (i, 0)),
            pl.BlockSpec((d, cb), lambda i, j: (0, j)),
            pl.BlockSpec((1, d), lambda i, j: (0, 0)),
            pl.BlockSpec((1, d), lambda i, j: (0, 0)),
        ],
        out_specs=pl.BlockSpec((ln, cb), lambda i, j: (i, j)),
        out_shape=jax.ShapeDtypeStruct((n, 2 * di), f32),
    )(xf, in_proj_w.T, ln1_w.reshape(1, d), ln1_b.reshape(1, d))

    xs, dt, bp, cp = pl.pallas_call(
        _conv_body,
        grid=(n_nb,),
        in_specs=[
            pl.BlockSpec((ln, di), lambda i: (i, 0)),
            pl.BlockSpec((conv_w.shape[1], di), lambda i: (0, 0)),
            pl.BlockSpec((1, di), lambda i: (0, 0)),
            pl.BlockSpec((di, r + 2 * s), lambda i: (0, 0)),
            pl.BlockSpec((r, di), lambda i: (0, 0)),
            pl.BlockSpec((1, di), lambda i: (0, 0)),
        ],
        out_specs=[
            pl.BlockSpec((ln, di), lambda i: (i, 0)),
            pl.BlockSpec((ln, di), lambda i: (i, 0)),
            pl.BlockSpec((ln, s), lambda i: (i, 0)),
            pl.BlockSpec((ln, s), lambda i: (i, 0)),
        ],
        out_shape=[
            jax.ShapeDtypeStruct((n, di), f32),
            jax.ShapeDtypeStruct((n, di), f32),
            jax.ShapeDtypeStruct((n, s), f32),
            jax.ShapeDtypeStruct((n, s), f32),
        ],
        scratch_shapes=[pltpu.VMEM((8, di), f32)],
    )(xz, conv_w.T, conv_b.reshape(1, di), x_proj_w.T, dt_proj_w.T,
      dt_proj_b.reshape(1, di))

    lc = 256
    ys = pl.pallas_call(
        _scan_body,
        grid=(n // lc,),
        in_specs=[
            pl.BlockSpec((lc, di), lambda i: (i, 0)),
            pl.BlockSpec((lc, di), lambda i: (i, 0)),
            pl.BlockSpec((lc, s), lambda i: (i, 0)),
            pl.BlockSpec((lc, s), lambda i: (i, 0)),
            pl.BlockSpec((s, di), lambda i: (0, 0)),
            pl.BlockSpec((1, di), lambda i: (0, 0)),
        ],
        out_specs=pl.BlockSpec((lc, di), lambda i: (i, 0)),
        out_shape=jax.ShapeDtypeStruct((n, di), f32),
        scratch_shapes=[pltpu.VMEM((s, di), f32),
                        pltpu.VMEM((32, s, di), f32),
                        pltpu.VMEM((32, s, di), f32)],
    )(xs, dt, bp, cp, A_log.T, Dp.reshape(1, di))

    h, hn, comb, ints12, w12 = pl.pallas_call(
        _post_body,
        grid=(n_nb,),
        in_specs=[
            pl.BlockSpec((ln, di), lambda i: (i, 0)),
            pl.BlockSpec((ln, di), lambda i: (i, 1)),
            pl.BlockSpec((di, d), lambda i: (0, 0)),
            pl.BlockSpec((ln, d), lambda i: (i, 0)),
            pl.BlockSpec((1, d), lambda i: (0, 0)),
            pl.BlockSpec((1, d), lambda i: (0, 0)),
            pl.BlockSpec((d, e), lambda i: (0, 0)),
        ],
        out_specs=[
            pl.BlockSpec((ln, d), lambda i: (i, 0)),
            pl.BlockSpec((ln, d), lambda i: (i, 0)),
            pl.BlockSpec((ln, e), lambda i: (i, 0)),
            pl.BlockSpec((ln, 2), lambda i: (i, 0)),
            pl.BlockSpec((ln, 2), lambda i: (i, 0)),
        ],
        out_shape=[
            jax.ShapeDtypeStruct((n, d), f32),
            jax.ShapeDtypeStruct((n, d), f32),
            jax.ShapeDtypeStruct((n, e), f32),
            jax.ShapeDtypeStruct((n, 2), jnp.int32),
            jax.ShapeDtypeStruct((n, 2), f32),
        ],
    )(ys, xz, out_proj_w.T, xf, ln2_w.reshape(1, d), ln2_b.reshape(1, d),
      router_w.T)

    p_max = 2 * n + e * _RBLK
    nq = p_max // _RBLK
    dest12, beq = pl.pallas_call(
        _route_body,
        grid=(1,),
        in_specs=[
            pl.BlockSpec((n, e), lambda i: (0, 0)),
            pl.BlockSpec((n, 2), lambda i: (0, 0)),
        ],
        out_specs=[
            pl.BlockSpec((n, 2), lambda i: (0, 0)),
            pl.BlockSpec((_RBLK, e), lambda i: (0, 0)),
        ],
        out_shape=[
            jax.ShapeDtypeStruct((n, 2), jnp.int32),
            jax.ShapeDtypeStruct((_RBLK, e), jnp.int32),
        ],
    )(comb, ints12)
    be = beq[:nq, 0]
    dflat = jnp.concatenate([dest12[:, 0], dest12[:, 1]])
    tokflat = jnp.concatenate([jnp.arange(n, dtype=jnp.int32)] * 2)

    nw = 32
    mesh = plsc.VectorSubcoreMesh(core_axis_name="c", subcore_axis_name="s")
    a_per_w = (2 * n) // nw

    @functools.partial(
        pl.kernel, mesh=mesh,
        out_type=jax.ShapeDtypeStruct((p_max,), jnp.int32),
        scratch_types=[
            pltpu.VMEM((a_per_w,), jnp.int32),
            pltpu.VMEM((a_per_w,), jnp.int32),
            pltpu.SemaphoreType.DMA,
        ],
    )
    def _sc_scatter(dflat_hbm, tok_hbm, src_hbm, idx_v, val_v, sem):
        wid = lax.axis_index("s") * 2 + lax.axis_index("c")
        base = wid * a_per_w
        pltpu.sync_copy(dflat_hbm.at[pl.ds(base, a_per_w)], idx_v)
        pltpu.sync_copy(tok_hbm.at[pl.ds(base, a_per_w)], val_v)
        pltpu.async_copy(val_v, src_hbm.at[idx_v], sem).wait()

    src = _sc_scatter(dflat, tokflat)

    s_per_w = p_max // nw
    s_chunk = s_per_w // 2

    @functools.partial(
        pl.kernel, mesh=mesh,
        out_type=jax.ShapeDtypeStruct((p_max, d), f32),
        scratch_types=[
            pltpu.VMEM((s_chunk,), jnp.int32),
            pltpu.VMEM((s_chunk, d), f32),
            pltpu.SemaphoreType.DMA,
        ],
    )
    def _sc_gather(src_hbm, hn_hbm, out_hbm, idx_v, rows_v, sem):
        wid = lax.axis_index("s") * 2 + lax.axis_index("c")
        base = wid * s_per_w
        for c in range(2):
            pltpu.sync_copy(src_hbm.at[pl.ds(base + c * s_chunk, s_chunk)], idx_v)
            for i in range(s_chunk // 16):
                v = idx_v[pl.ds(i * 16, 16)]
                idx_v[pl.ds(i * 16, 16)] = jnp.minimum(jnp.maximum(v, 0), n - 1)
            pltpu.async_copy(hn_hbm.at[idx_v], rows_v, sem).wait()
            pltpu.sync_copy(rows_v, out_hbm.at[pl.ds(base + c * s_chunk, s_chunk)])

    xs_sorted = _sc_gather(src, hn)

    bh = 512
    eo_sorted = pl.pallas_call(
        _gmm_body,
        grid_spec=pltpu.PrefetchScalarGridSpec(
            num_scalar_prefetch=1,
            grid=(nq, h_dim // bh),
            in_specs=[
                pl.BlockSpec((_RBLK, d), lambda q, k, be_r: (q, 0)),
                pl.BlockSpec((1, bh, d), lambda q, k, be_r: (be_r[q], k, 0)),
                pl.BlockSpec((1, bh, d), lambda q, k, be_r: (be_r[q], k, 0)),
                pl.BlockSpec((1, d, bh), lambda q, k, be_r: (be_r[q], 0, k)),
            ],
            out_specs=pl.BlockSpec((_RBLK, d), lambda q, k, be_r: (q, 0)),
        ),
        out_shape=jax.ShapeDtypeStruct((p_max, d), f32),
    )(be, xs_sorted, w_gate, w_up, w_down)

    out = pl.pallas_call(
        _combine_body,
        grid=(n // _RBLK,),
        in_specs=[
            pl.BlockSpec((p_max, d), lambda i: (0, 0)),
            pl.BlockSpec((_RBLK, d), lambda i: (i, 0)),
            pl.BlockSpec(memory_space=pltpu.SMEM),
            pl.BlockSpec(memory_space=pltpu.SMEM),
        ],
        out_specs=pl.BlockSpec((_RBLK, d), lambda i: (i, 0)),
        out_shape=jax.ShapeDtypeStruct((n, d), f32),
    )(eo_sorted, h, dflat, jnp.concatenate([w12[:, 0], w12[:, 1]]))

    return out.reshape(b, t, d)


# speedup vs baseline: 1.1410x; 1.1410x over previous
import functools

import jax
import jax.numpy as jnp
from jax import lax
from jax.experimental import pallas as pl
from jax.experimental.pallas import tpu as pltpu
from jax.experimental.pallas import tpu_sc as plsc


def _silu(v):
    return v * jax.nn.sigmoid(v)


def _ln(v, w, b):
    m = v.mean(-1, keepdims=True)
    var = ((v - m) ** 2).mean(-1, keepdims=True)
    return (v - m) * jax.lax.rsqrt(var + 1e-5) * w + b


def _inproj_body(x_ref, w_ref, lnw_ref, lnb_ref, o_ref):
    xn = _ln(x_ref[...], lnw_ref[...], lnb_ref[...])
    o_ref[...] = jnp.dot(xn, w_ref[...], preferred_element_type=jnp.float32)


def _conv_body(xin_ref, convw_ref, convb_ref, xpw_ref, dtw_ref, dtb_ref,
               xs_ref, dt_ref, bp_ref, cp_ref, carry_ref):
    nb = pl.program_id(0)

    @pl.when(nb == 0)
    def _():
        carry_ref[...] = jnp.zeros_like(carry_ref)

    xin = xin_ref[...]
    ln = xin.shape[0]
    dc = convw_ref.shape[0]
    ext = jnp.concatenate([carry_ref[...], xin], axis=0)
    acc = jnp.broadcast_to(convb_ref[...], xin.shape)
    for k in range(dc):
        acc = acc + convw_ref[k, :][None, :] * ext[8 + k - (dc - 1): 8 + k - (dc - 1) + ln, :]
    xs = _silu(acc)
    xs_ref[...] = xs
    carry_ref[...] = xin[ln - 8: ln, :]
    xp = jnp.dot(xs, xpw_ref[...], preferred_element_type=jnp.float32)
    r = dtw_ref.shape[0]
    s = bp_ref.shape[1]
    bp_ref[...] = xp[:, r: r + s]
    cp_ref[...] = xp[:, r + s: r + 2 * s]
    dt_ref[...] = jax.nn.softplus(
        jnp.dot(xp[:, :r], dtw_ref[...], preferred_element_type=jnp.float32)
        + dtb_ref[...])


def _scan_body(xs_ref, dt_ref, bp_ref, cp_ref, alogt_ref, dp_ref, y_ref,
               h_ref, da_ref, dbx_ref):
    g = pl.program_id(0)

    @pl.when(g == 0)
    def _():
        h_ref[...] = jnp.zeros_like(h_ref)

    a_neg = -jnp.exp(alogt_ref[...])
    dp = dp_ref[...]
    lc = xs_ref.shape[0]
    u = 32
    outer_dn = (((0,), (0,)), ((), ()))
    contr_dn = (((1,), (0,)), ((), ()))

    def body(k, h):
        j0 = pl.multiple_of(k * u, u)
        dt_c = dt_ref[pl.ds(j0, u), :]
        xs_c = xs_ref[pl.ds(j0, u), :]
        bp_c = bp_ref[pl.ds(j0, u), :]
        cp_c = cp_ref[pl.ds(j0, u), :]
        dtx_c = dt_c * xs_c
        for j in range(u):
            da_ref[j] = jnp.exp(dt_c[j:j + 1, :] * a_neg)
            dbx_ref[j] = jax.lax.dot_general(
                bp_c[j:j + 1, :], dtx_c[j:j + 1, :], outer_dn,
                preferred_element_type=jnp.float32)
        rows = []
        for j in range(u):
            h = da_ref[j] * h + dbx_ref[j]
            rows.append(jax.lax.dot_general(cp_c[j:j + 1, :], h, contr_dn,
                                            preferred_element_type=jnp.float32))
        y_ref[pl.ds(j0, u), :] = jnp.concatenate(rows, axis=0) + dp * xs_c
        return h

    h_ref[...] = jax.lax.fori_loop(0, lc // u, body, h_ref[...])


def _post_body(ys_ref, z_ref, opw_ref, x_ref, ln2w_ref, ln2b_ref, rw_ref,
               h_ref, hn_ref, comb_ref, ints_ref, w12_ref):
    y = ys_ref[...] * _silu(z_ref[...])
    h = x_ref[...] + jnp.dot(y, opw_ref[...], preferred_element_type=jnp.float32)
    h_ref[...] = h
    hn = _ln(h, ln2w_ref[...], ln2b_ref[...])
    hn_ref[...] = hn
    logits = jnp.dot(hn, rw_ref[...], preferred_element_type=jnp.float32)
    p = jax.nn.softmax(logits, axis=-1)
    col = jax.lax.broadcasted_iota(jnp.int32, p.shape, 1)
    i1 = jnp.argmax(p, axis=-1, keepdims=True)
    m1 = jnp.max(p, axis=-1, keepdims=True)
    pm = jnp.where(col == i1, -jnp.inf, p)
    i2 = jnp.argmax(pm, axis=-1, keepdims=True)
    m2 = jnp.max(pm, axis=-1, keepdims=True)
    tot = m1 + m2
    comb_ref[...] = jnp.where(col == i1, m1 / tot,
                              jnp.where(col == i2, m2 / tot, 0.0))
    ints_ref[...] = jnp.concatenate([i1, i2], axis=1).astype(jnp.int32)
    w12_ref[...] = jnp.concatenate([m1 / tot, m2 / tot], axis=1)


_RBLK = 128


def _route_body(comb_ref, ints_ref, dest_ref, beq_ref):
    n, e = comb_ref.shape
    nbk = n // _RBLK
    f32 = jnp.float32
    m = (comb_ref[...] != 0.0).astype(f32)
    ri = lax.broadcasted_iota(jnp.int32, (_RBLK, _RBLK), 0)
    ci = lax.broadcasted_iota(jnp.int32, (_RBLK, _RBLK), 1)
    sl = (ci < ri).astype(f32)
    ones_row = jnp.ones((1, _RBLK), f32)
    l16 = (lax.broadcasted_iota(jnp.int32, (nbk, nbk), 1)
           <= lax.broadcasted_iota(jnp.int32, (nbk, nbk), 0)).astype(f32)
    su8 = (lax.broadcasted_iota(jnp.int32, (e, e), 0)
           < lax.broadcasted_iota(jnp.int32, (e, e), 1)).astype(f32)
    dnum = (((1,), (0,)), ((), ()))
    ranks = []
    cnts = []
    for b in range(nbk):
        mb = m[b * _RBLK:(b + 1) * _RBLK, :]
        ranks.append(lax.dot_general(sl, mb, dnum, preferred_element_type=f32))
        cnts.append(lax.dot_general(ones_row, mb, dnum, preferred_element_type=f32))
    pbc = jnp.concatenate(cnts, axis=0)
    p_incl = lax.dot_general(l16, pbc, dnum, preferred_element_type=f32)
    tot = p_incl[nbk - 1:nbk, :]
    ti = tot.astype(jnp.int32)
    ptf = (((ti + (_RBLK - 1)) >> 7) << 7).astype(f32)
    pbase = lax.dot_general(ptf, su8, dnum, preferred_element_type=f32)
    bot = pbase + p_incl - pbc
    col8 = lax.broadcasted_iota(jnp.int32, (_RBLK, e), 1)
    rows = []
    for b in range(nbk):
        dest_b = ranks[b] + bot[b:b + 1, :]
        i1 = ints_ref[b * _RBLK:(b + 1) * _RBLK, 0:1]
        i2 = ints_ref[b * _RBLK:(b + 1) * _RBLK, 1:2]
        d1 = jnp.sum(jnp.where(col8 == i1, dest_b, 0.0), axis=1, keepdims=True)
        d2 = jnp.sum(jnp.where(col8 == i2, dest_b, 0.0), axis=1, keepdims=True)
        rows.append(jnp.concatenate([d1, d2], axis=1))
    dest_ref[...] = jnp.concatenate(rows, axis=0).astype(jnp.int32)
    qlo = lax.broadcasted_iota(jnp.int32, beq_ref.shape, 0).astype(f32) * _RBLK
    ecol = lax.broadcasted_iota(jnp.int32, beq_ref.shape, 1).astype(f32)
    inseg = (qlo >= pbase) & (qlo < pbase + ptf)
    be_col = jnp.sum(jnp.where(inseg, ecol, 0.0), axis=1, keepdims=True)
    beq_ref[...] = jnp.broadcast_to(be_col, beq_ref.shape).astype(jnp.int32)


def _gmm_body(be_ref, xr_ref, wg_ref, wu_ref, wd_ref, o_ref):
    hb = pl.program_id(1)

    @pl.when(hb == 0)
    def _():
        o_ref[...] = jnp.zeros_like(o_ref)

    xb = xr_ref[...].astype(jnp.bfloat16)
    dn = (((1,), (1,)), ((), ()))
    wg = wg_ref[0].astype(jnp.bfloat16)
    wu = wu_ref[0].astype(jnp.bfloat16)
    wd = wd_ref[0].astype(jnp.bfloat16)
    g = lax.dot_general(xb, wg, dn, preferred_element_type=jnp.float32)
    u = lax.dot_general(xb, wu, dn, preferred_element_type=jnp.float32)
    act = (_silu(g) * u).astype(jnp.bfloat16)
    o_ref[...] += lax.dot_general(act, wd, dn, preferred_element_type=jnp.float32)


def _combine_body(eo_ref, h_ref, dest_ref, w_ref, o_ref):
    blk = pl.program_id(0)
    bn = h_ref.shape[0]
    n = bn * pl.num_programs(0)

    def body(j, carry):
        tok = blk * bn + j
        d1 = dest_ref[tok]
        d2 = dest_ref[n + tok]
        w1 = w_ref[tok]
        w2 = w_ref[n + tok]
        o_ref[pl.ds(j, 1), :] = (h_ref[pl.ds(j, 1), :]
                                 + w1 * eo_ref[pl.ds(d1, 1), :]
                                 + w2 * eo_ref[pl.ds(d2, 1), :])
        return carry

    lax.fori_loop(0, bn, body, 0)


def kernel(x, ln1_w, ln1_b, ln2_w, ln2_b, in_proj_w, conv_w, conv_b, x_proj_w,
           dt_proj_w, dt_proj_b, A_log, Dp, out_proj_w, router_w, w_gate, w_up, w_down):
    b, t, d = x.shape
    di = conv_w.shape[0]
    r = dt_proj_w.shape[1]
    s = A_log.shape[1]
    e = router_w.shape[0]
    h_dim = w_gate.shape[1]
    n = b * t
    f32 = jnp.float32
    xf = x.reshape(n, d)

    ln = 256
    cb = 768
    n_nb = n // ln

    xz = pl.pallas_call(
        _inproj_body,
        grid=(n_nb, (2 * di) // cb),
        in_specs=[
            pl.BlockSpec((ln, d), lambda i, j: (i, 0)),
            pl.BlockSpec((d, cb), lambda i, j: (0, j)),
            pl.BlockSpec((1, d), lambda i, j: (0, 0)),
            pl.BlockSpec((1, d), lambda i, j: (0, 0)),
        ],
        out_specs=pl.BlockSpec((ln, cb), lambda i, j: (i, j)),
        out_shape=jax.ShapeDtypeStruct((n, 2 * di), f32),
    )(xf, in_proj_w.T, ln1_w.reshape(1, d), ln1_b.reshape(1, d))

    xs, dt, bp, cp = pl.pallas_call(
        _conv_body,
        grid=(n_nb,),
        in_specs=[
            pl.BlockSpec((ln, di), lambda i: (i, 0)),
            pl.BlockSpec((conv_w.shape[1], di), lambda i: (0, 0)),
            pl.BlockSpec((1, di), lambda i: (0, 0)),
            pl.BlockSpec((di, r + 2 * s), lambda i: (0, 0)),
            pl.BlockSpec((r, di), lambda i: (0, 0)),
            pl.BlockSpec((1, di), lambda i: (0, 0)),
        ],
        out_specs=[
            pl.BlockSpec((ln, di), lambda i: (i, 0)),
            pl.BlockSpec((ln, di), lambda i: (i, 0)),
            pl.BlockSpec((ln, s), lambda i: (i, 0)),
            pl.BlockSpec((ln, s), lambda i: (i, 0)),
        ],
        out_shape=[
            jax.ShapeDtypeStruct((n, di), f32),
            jax.ShapeDtypeStruct((n, di), f32),
            jax.ShapeDtypeStruct((n, s), f32),
            jax.ShapeDtypeStruct((n, s), f32),
        ],
        scratch_shapes=[pltpu.VMEM((8, di), f32)],
    )(xz, conv_w.T, conv_b.reshape(1, di), x_proj_w.T, dt_proj_w.T,
      dt_proj_b.reshape(1, di))

    lc = 256
    ys = pl.pallas_call(
        _scan_body,
        grid=(n // lc,),
        in_specs=[
            pl.BlockSpec((lc, di), lambda i: (i, 0)),
            pl.BlockSpec((lc, di), lambda i: (i, 0)),
            pl.BlockSpec((lc, s), lambda i: (i, 0)),
            pl.BlockSpec((lc, s), lambda i: (i, 0)),
            pl.BlockSpec((s, di), lambda i: (0, 0)),
            pl.BlockSpec((1, di), lambda i: (0, 0)),
        ],
        out_specs=pl.BlockSpec((lc, di), lambda i: (i, 0)),
        out_shape=jax.ShapeDtypeStruct((n, di), f32),
        scratch_shapes=[pltpu.VMEM((s, di), f32),
                        pltpu.VMEM((32, s, di), f32),
                        pltpu.VMEM((32, s, di), f32)],
    )(xs, dt, bp, cp, A_log.T, Dp.reshape(1, di))

    h, hn, comb, ints12, w12 = pl.pallas_call(
        _post_body,
        grid=(n_nb,),
        in_specs=[
            pl.BlockSpec((ln, di), lambda i: (i, 0)),
            pl.BlockSpec((ln, di), lambda i: (i, 1)),
            pl.BlockSpec((di, d), lambda i: (0, 0)),
            pl.BlockSpec((ln, d), lambda i: (i, 0)),
            pl.BlockSpec((1, d), lambda i: (0, 0)),
            pl.BlockSpec((1, d), lambda i: (0, 0)),
            pl.BlockSpec((d, e), lambda i: (0, 0)),
        ],
        out_specs=[
            pl.BlockSpec((ln, d), lambda i: (i, 0)),
            pl.BlockSpec((ln, d), lambda i: (i, 0)),
            pl.BlockSpec((ln, e), lambda i: (i, 0)),
            pl.BlockSpec((ln, 2), lambda i: (i, 0)),
            pl.BlockSpec((ln, 2), lambda i: (i, 0)),
        ],
        out_shape=[
            jax.ShapeDtypeStruct((n, d), f32),
            jax.ShapeDtypeStruct((n, d), f32),
            jax.ShapeDtypeStruct((n, e), f32),
            jax.ShapeDtypeStruct((n, 2), jnp.int32),
            jax.ShapeDtypeStruct((n, 2), f32),
        ],
    )(ys, xz, out_proj_w.T, xf, ln2_w.reshape(1, d), ln2_b.reshape(1, d),
      router_w.T)

    p_max = 2 * n + e * _RBLK
    nq = p_max // _RBLK
    dest12, beq = pl.pallas_call(
        _route_body,
        grid=(1,),
        in_specs=[
            pl.BlockSpec((n, e), lambda i: (0, 0)),
            pl.BlockSpec((n, 2), lambda i: (0, 0)),
        ],
        out_specs=[
            pl.BlockSpec((n, 2), lambda i: (0, 0)),
            pl.BlockSpec((_RBLK, e), lambda i: (0, 0)),
        ],
        out_shape=[
            jax.ShapeDtypeStruct((n, 2), jnp.int32),
            jax.ShapeDtypeStruct((_RBLK, e), jnp.int32),
        ],
    )(comb, ints12)
    be = beq[:nq, 0]
    dflat = jnp.concatenate([dest12[:, 0], dest12[:, 1]])

    nw = 32
    mesh = plsc.VectorSubcoreMesh(core_axis_name="c", subcore_axis_name="s")
    a_per_w = (2 * n) // nw

    @functools.partial(
        pl.kernel, mesh=mesh,
        out_type=jax.ShapeDtypeStruct((p_max, d), f32),
        scratch_types=[
            pltpu.VMEM((a_per_w,), jnp.int32),
            pltpu.VMEM((a_per_w, d), f32),
            pltpu.SemaphoreType.DMA,
        ],
    )
    def _sc_disperse(dflat_hbm, hn_hbm, out_hbm, idx_v, rows_v, sem):
        wid = lax.axis_index("s") * 2 + lax.axis_index("c")
        pltpu.sync_copy(dflat_hbm.at[pl.ds(wid * a_per_w, a_per_w)], idx_v)
        tok0 = (wid % 16) * a_per_w
        pltpu.sync_copy(hn_hbm.at[pl.ds(tok0, a_per_w)], rows_v)
        pltpu.async_copy(rows_v, out_hbm.at[idx_v], sem).wait()

    xs_sorted = _sc_disperse(dflat, hn)

    bh = 512
    eo_sorted = pl.pallas_call(
        _gmm_body,
        grid_spec=pltpu.PrefetchScalarGridSpec(
            num_scalar_prefetch=1,
            grid=(nq, h_dim // bh),
            in_specs=[
                pl.BlockSpec((_RBLK, d), lambda q, k, be_r: (q, 0)),
                pl.BlockSpec((1, bh, d), lambda q, k, be_r: (be_r[q], k, 0)),
                pl.BlockSpec((1, bh, d), lambda q, k, be_r: (be_r[q], k, 0)),
                pl.BlockSpec((1, d, bh), lambda q, k, be_r: (be_r[q], 0, k)),
            ],
            out_specs=pl.BlockSpec((_RBLK, d), lambda q, k, be_r: (q, 0)),
        ),
        out_shape=jax.ShapeDtypeStruct((p_max, d), f32),
    )(be, xs_sorted, w_gate, w_up, w_down)

    out = pl.pallas_call(
        _combine_body,
        grid=(n // _RBLK,),
        in_specs=[
            pl.BlockSpec((p_max, d), lambda i: (0, 0)),
            pl.BlockSpec((_RBLK, d), lambda i: (i, 0)),
            pl.BlockSpec(memory_space=pltpu.SMEM),
            pl.BlockSpec(memory_space=pltpu.SMEM),
        ],
        out_specs=pl.BlockSpec((_RBLK, d), lambda i: (i, 0)),
        out_shape=jax.ShapeDtypeStruct((n, d), f32),
    )(eo_sorted, h, dflat, jnp.concatenate([w12[:, 0], w12[:, 1]]))

    return out.reshape(b, t, d)


# speedup vs baseline: 1.1527x; 1.0102x over previous
import functools

import jax
import jax.numpy as jnp
from jax import lax
from jax.experimental import pallas as pl
from jax.experimental.pallas import tpu as pltpu
from jax.experimental.pallas import tpu_sc as plsc


def _silu(v):
    return v * jax.nn.sigmoid(v)


def _ln(v, w, b):
    m = v.mean(-1, keepdims=True)
    var = ((v - m) ** 2).mean(-1, keepdims=True)
    return (v - m) * jax.lax.rsqrt(var + 1e-5) * w + b


def _inproj_body(x_ref, w_ref, lnw_ref, lnb_ref, o_ref):
    xn = _ln(x_ref[...], lnw_ref[...], lnb_ref[...])
    o_ref[...] = jnp.dot(xn, w_ref[...], preferred_element_type=jnp.float32)


def _conv_body(xin_ref, convw_ref, convb_ref, xpw_ref, dtw_ref, dtb_ref,
               xs_ref, dt_ref, bp_ref, cp_ref, carry_ref):
    nb = pl.program_id(0)

    @pl.when(nb == 0)
    def _():
        carry_ref[...] = jnp.zeros_like(carry_ref)

    xin = xin_ref[...]
    ln = xin.shape[0]
    dc = convw_ref.shape[0]
    ext = jnp.concatenate([carry_ref[...], xin], axis=0)
    acc = jnp.broadcast_to(convb_ref[...], xin.shape)
    for k in range(dc):
        acc = acc + convw_ref[k, :][None, :] * ext[8 + k - (dc - 1): 8 + k - (dc - 1) + ln, :]
    xs = _silu(acc)
    xs_ref[...] = xs
    carry_ref[...] = xin[ln - 8: ln, :]
    xp = jnp.dot(xs, xpw_ref[...], preferred_element_type=jnp.float32)
    r = dtw_ref.shape[0]
    s = bp_ref.shape[1]
    bp_ref[...] = xp[:, r: r + s]
    cp_ref[...] = xp[:, r + s: r + 2 * s]
    dt_ref[...] = jax.nn.softplus(
        jnp.dot(xp[:, :r], dtw_ref[...], preferred_element_type=jnp.float32)
        + dtb_ref[...])


def _scan_body(xs_ref, dt_ref, bp_ref, cp_ref, alogt_ref, dp_ref, y_ref,
               h_ref, da_ref, dbx_ref):
    g = pl.program_id(0)

    @pl.when(g == 0)
    def _():
        h_ref[...] = jnp.zeros_like(h_ref)

    a_neg = -jnp.exp(alogt_ref[...])
    dp = dp_ref[...]
    lc = xs_ref.shape[0]
    u = 64
    outer_dn = (((0,), (0,)), ((), ()))
    contr_dn = (((1,), (0,)), ((), ()))

    def body(k, h):
        j0 = pl.multiple_of(k * u, u)
        dt_c = dt_ref[pl.ds(j0, u), :]
        xs_c = xs_ref[pl.ds(j0, u), :]
        bp_c = bp_ref[pl.ds(j0, u), :]
        cp_c = cp_ref[pl.ds(j0, u), :]
        dtx_c = dt_c * xs_c
        for j in range(u):
            da_ref[j] = jnp.exp(dt_c[j:j + 1, :] * a_neg)
            dbx_ref[j] = jax.lax.dot_general(
                bp_c[j:j + 1, :], dtx_c[j:j + 1, :], outer_dn,
                preferred_element_type=jnp.float32)
        rows = []
        for j in range(u):
            h = da_ref[j] * h + dbx_ref[j]
            rows.append(jax.lax.dot_general(cp_c[j:j + 1, :], h, contr_dn,
                                            preferred_element_type=jnp.float32))
        y_ref[pl.ds(j0, u), :] = jnp.concatenate(rows, axis=0) + dp * xs_c
        return h

    h_ref[...] = jax.lax.fori_loop(0, lc // u, body, h_ref[...])


def _post_body(ys_ref, z_ref, opw_ref, x_ref, ln2w_ref, ln2b_ref, rw_ref,
               h_ref, hn_ref, comb_ref, ints_ref, w12_ref):
    y = ys_ref[...] * _silu(z_ref[...])
    h = x_ref[...] + jnp.dot(y, opw_ref[...], preferred_element_type=jnp.float32)
    h_ref[...] = h
    hn = _ln(h, ln2w_ref[...], ln2b_ref[...])
    hn_ref[...] = hn
    logits = jnp.dot(hn, rw_ref[...], preferred_element_type=jnp.float32)
    p = jax.nn.softmax(logits, axis=-1)
    col = jax.lax.broadcasted_iota(jnp.int32, p.shape, 1)
    i1 = jnp.argmax(p, axis=-1, keepdims=True)
    m1 = jnp.max(p, axis=-1, keepdims=True)
    pm = jnp.where(col == i1, -jnp.inf, p)
    i2 = jnp.argmax(pm, axis=-1, keepdims=True)
    m2 = jnp.max(pm, axis=-1, keepdims=True)
    tot = m1 + m2
    comb_ref[...] = jnp.where(col == i1, m1 / tot,
                              jnp.where(col == i2, m2 / tot, 0.0))
    ints_ref[...] = jnp.concatenate([i1, i2], axis=1).astype(jnp.int32)
    w12_ref[...] = jnp.concatenate([m1 / tot, m2 / tot], axis=1)


_RBLK = 128


def _route_body(comb_ref, ints_ref, dest_ref, w_ref, beq_ref):
    n, e = comb_ref.shape
    nbk = n // _RBLK
    f32 = jnp.float32
    m = (comb_ref[...] != 0.0).astype(f32)
    ri = lax.broadcasted_iota(jnp.int32, (_RBLK, _RBLK), 0)
    ci = lax.broadcasted_iota(jnp.int32, (_RBLK, _RBLK), 1)
    sl = (ci < ri).astype(f32)
    ones_row = jnp.ones((1, _RBLK), f32)
    l16 = (lax.broadcasted_iota(jnp.int32, (nbk, nbk), 1)
           <= lax.broadcasted_iota(jnp.int32, (nbk, nbk), 0)).astype(f32)
    su8 = (lax.broadcasted_iota(jnp.int32, (e, e), 0)
           < lax.broadcasted_iota(jnp.int32, (e, e), 1)).astype(f32)
    dnum = (((1,), (0,)), ((), ()))
    ranks = []
    cnts = []
    for b in range(nbk):
        mb = m[b * _RBLK:(b + 1) * _RBLK, :]
        ranks.append(lax.dot_general(sl, mb, dnum, preferred_element_type=f32))
        cnts.append(lax.dot_general(ones_row, mb, dnum, preferred_element_type=f32))
    pbc = jnp.concatenate(cnts, axis=0)
    p_incl = lax.dot_general(l16, pbc, dnum, preferred_element_type=f32)
    tot = p_incl[nbk - 1:nbk, :]
    ti = tot.astype(jnp.int32)
    ptf = (((ti + (_RBLK - 1)) >> 7) << 7).astype(f32)
    pbase = lax.dot_general(ptf, su8, dnum, preferred_element_type=f32)
    bot = pbase + p_incl - pbc
    col8 = lax.broadcasted_iota(jnp.int32, (_RBLK, e), 1)
    for b in range(nbk):
        dest_b = ranks[b] + bot[b:b + 1, :]
        cb = comb_ref[b * _RBLK:(b + 1) * _RBLK, :]
        i1 = ints_ref[b * _RBLK:(b + 1) * _RBLK, 0:1]
        i2 = ints_ref[b * _RBLK:(b + 1) * _RBLK, 1:2]
        lo = b * _RBLK
        hi = (b + 1) * _RBLK
        for idx, (dlo, dhi) in ((i1, (lo, hi)), (i2, (n + lo, n + hi))):
            sel = col8 == idx
            dest_ref[dlo:dhi, :] = jnp.sum(
                jnp.where(sel, dest_b, 0.0), axis=1, keepdims=True).astype(jnp.int32)
            w_ref[dlo:dhi, :] = jnp.sum(
                jnp.where(sel, cb, 0.0), axis=1, keepdims=True)
    qlo = lax.broadcasted_iota(jnp.int32, beq_ref.shape, 0).astype(f32) * _RBLK
    ecol = lax.broadcasted_iota(jnp.int32, beq_ref.shape, 1).astype(f32)
    inseg = (qlo >= pbase) & (qlo < pbase + ptf)
    be_col = jnp.sum(jnp.where(inseg, ecol, 0.0), axis=1, keepdims=True)
    beq_ref[...] = jnp.broadcast_to(be_col, beq_ref.shape).astype(jnp.int32)


def _gmm_body(be_ref, xr_ref, wg_ref, wu_ref, wd_ref, o_ref):
    hb = pl.program_id(1)

    @pl.when(hb == 0)
    def _():
        o_ref[...] = jnp.zeros_like(o_ref)

    xb = xr_ref[...].astype(jnp.bfloat16)
    dn = (((1,), (1,)), ((), ()))
    wg = wg_ref[0].astype(jnp.bfloat16)
    wu = wu_ref[0].astype(jnp.bfloat16)
    wd = wd_ref[0].astype(jnp.bfloat16)
    g = lax.dot_general(xb, wg, dn, preferred_element_type=jnp.float32)
    u = lax.dot_general(xb, wu, dn, preferred_element_type=jnp.float32)
    act = (_silu(g) * u).astype(jnp.bfloat16)
    o_ref[...] += lax.dot_general(act, wd, dn, preferred_element_type=jnp.float32)


def _combine_body(eo_ref, h_ref, dest_ref, w_ref, o_ref):
    blk = pl.program_id(0)
    bn = h_ref.shape[0]
    n = bn * pl.num_programs(0)

    def body(j, carry):
        tok = blk * bn + j
        d1 = dest_ref[tok]
        d2 = dest_ref[n + tok]
        w1 = w_ref[tok]
        w2 = w_ref[n + tok]
        o_ref[pl.ds(j, 1), :] = (h_ref[pl.ds(j, 1), :]
                                 + w1 * eo_ref[pl.ds(d1, 1), :]
                                 + w2 * eo_ref[pl.ds(d2, 1), :])
        return carry

    lax.fori_loop(0, bn, body, 0)


def kernel(x, ln1_w, ln1_b, ln2_w, ln2_b, in_proj_w, conv_w, conv_b, x_proj_w,
           dt_proj_w, dt_proj_b, A_log, Dp, out_proj_w, router_w, w_gate, w_up, w_down):
    b, t, d = x.shape
    di = conv_w.shape[0]
    r = dt_proj_w.shape[1]
    s = A_log.shape[1]
    e = router_w.shape[0]
    h_dim = w_gate.shape[1]
    n = b * t
    f32 = jnp.float32
    xf = x.reshape(n, d)

    ln = 256
    cb = 768
    n_nb = n // ln

    xz = pl.pallas_call(
        _inproj_body,
        grid=(n_nb, (2 * di) // cb),
        in_specs=[
            pl.BlockSpec((ln, d), lambda i, j: (i, 0)),
            pl.BlockSpec((d, cb), lambda i, j: (0, j)),
            pl.BlockSpec((1, d), lambda i, j: (0, 0)),
            pl.BlockSpec((1, d), lambda i, j: (0, 0)),
        ],
        out_specs=pl.BlockSpec((ln, cb), lambda i, j: (i, j)),
        out_shape=jax.ShapeDtypeStruct((n, 2 * di), f32),
    )(xf, in_proj_w.T, ln1_w.reshape(1, d), ln1_b.reshape(1, d))

    xs, dt, bp, cp = pl.pallas_call(
        _conv_body,
        grid=(n_nb,),
        in_specs=[
            pl.BlockSpec((ln, di), lambda i: (i, 0)),
            pl.BlockSpec((conv_w.shape[1], di), lambda i: (0, 0)),
            pl.BlockSpec((1, di), lambda i: (0, 0)),
            pl.BlockSpec((di, r + 2 * s), lambda i: (0, 0)),
            pl.BlockSpec((r, di), lambda i: (0, 0)),
            pl.BlockSpec((1, di), lambda i: (0, 0)),
        ],
        out_specs=[
            pl.BlockSpec((ln, di), lambda i: (i, 0)),
            pl.BlockSpec((ln, di), lambda i: (i, 0)),
            pl.BlockSpec((ln, s), lambda i: (i, 0)),
            pl.BlockSpec((ln, s), lambda i: (i, 0)),
        ],
        out_shape=[
            jax.ShapeDtypeStruct((n, di), f32),
            jax.ShapeDtypeStruct((n, di), f32),
            jax.ShapeDtypeStruct((n, s), f32),
            jax.ShapeDtypeStruct((n, s), f32),
        ],
        scratch_shapes=[pltpu.VMEM((8, di), f32)],
    )(xz, conv_w.T, conv_b.reshape(1, di), x_proj_w.T, dt_proj_w.T,
      dt_proj_b.reshape(1, di))

    lc = 256
    ys = pl.pallas_call(
        _scan_body,
        grid=(n // lc,),
        in_specs=[
            pl.BlockSpec((lc, di), lambda i: (i, 0)),
            pl.BlockSpec((lc, di), lambda i: (i, 0)),
            pl.BlockSpec((lc, s), lambda i: (i, 0)),
            pl.BlockSpec((lc, s), lambda i: (i, 0)),
            pl.BlockSpec((s, di), lambda i: (0, 0)),
            pl.BlockSpec((1, di), lambda i: (0, 0)),
        ],
        out_specs=pl.BlockSpec((lc, di), lambda i: (i, 0)),
        out_shape=jax.ShapeDtypeStruct((n, di), f32),
        scratch_shapes=[pltpu.VMEM((s, di), f32),
                        pltpu.VMEM((64, s, di), f32),
                        pltpu.VMEM((64, s, di), f32)],
    )(xs, dt, bp, cp, A_log.T, Dp.reshape(1, di))

    h, hn, comb, ints12, w12 = pl.pallas_call(
        _post_body,
        grid=(n_nb,),
        in_specs=[
            pl.BlockSpec((ln, di), lambda i: (i, 0)),
            pl.BlockSpec((ln, di), lambda i: (i, 1)),
            pl.BlockSpec((di, d), lambda i: (0, 0)),
            pl.BlockSpec((ln, d), lambda i: (i, 0)),
            pl.BlockSpec((1, d), lambda i: (0, 0)),
            pl.BlockSpec((1, d), lambda i: (0, 0)),
            pl.BlockSpec((d, e), lambda i: (0, 0)),
        ],
        out_specs=[
            pl.BlockSpec((ln, d), lambda i: (i, 0)),
            pl.BlockSpec((ln, d), lambda i: (i, 0)),
            pl.BlockSpec((ln, e), lambda i: (i, 0)),
            pl.BlockSpec((ln, 2), lambda i: (i, 0)),
            pl.BlockSpec((ln, 2), lambda i: (i, 0)),
        ],
        out_shape=[
            jax.ShapeDtypeStruct((n, d), f32),
            jax.ShapeDtypeStruct((n, d), f32),
            jax.ShapeDtypeStruct((n, e), f32),
            jax.ShapeDtypeStruct((n, 2), jnp.int32),
            jax.ShapeDtypeStruct((n, 2), f32),
        ],
    )(ys, xz, out_proj_w.T, xf, ln2_w.reshape(1, d), ln2_b.reshape(1, d),
      router_w.T)

    p_max = 2 * n + e * _RBLK
    nq = p_max // _RBLK
    destf, wf, beq = pl.pallas_call(
        _route_body,
        grid=(1,),
        in_specs=[
            pl.BlockSpec((n, e), lambda i: (0, 0)),
            pl.BlockSpec((n, 2), lambda i: (0, 0)),
        ],
        out_specs=[
            pl.BlockSpec((2 * n, 1), lambda i: (0, 0)),
            pl.BlockSpec((2 * n, 1), lambda i: (0, 0)),
            pl.BlockSpec((_RBLK, e), lambda i: (0, 0)),
        ],
        out_shape=[
            jax.ShapeDtypeStruct((2 * n, 1), jnp.int32),
            jax.ShapeDtypeStruct((2 * n, 1), f32),
            jax.ShapeDtypeStruct((_RBLK, e), jnp.int32),
        ],
    )(comb, ints12)
    be = beq[:nq, 0]
    dflat = destf.reshape(2 * n)
    wflat = wf.reshape(2 * n)

    nw = 32
    mesh = plsc.VectorSubcoreMesh(core_axis_name="c", subcore_axis_name="s")
    a_per_w = (2 * n) // nw

    @functools.partial(
        pl.kernel, mesh=mesh,
        out_type=jax.ShapeDtypeStruct((p_max, d), f32),
        scratch_types=[
            pltpu.VMEM((a_per_w,), jnp.int32),
            pltpu.VMEM((a_per_w, d), f32),
            pltpu.SemaphoreType.DMA,
        ],
    )
    def _sc_disperse(dflat_hbm, hn_hbm, out_hbm, idx_v, rows_v, sem):
        wid = lax.axis_index("s") * 2 + lax.axis_index("c")
        pltpu.sync_copy(dflat_hbm.at[pl.ds(wid * a_per_w, a_per_w)], idx_v)
        tok0 = (wid % 16) * a_per_w
        pltpu.sync_copy(hn_hbm.at[pl.ds(tok0, a_per_w)], rows_v)
        pltpu.async_copy(rows_v, out_hbm.at[idx_v], sem).wait()

    xs_sorted = _sc_disperse(dflat, hn)

    bh = 512
    eo_sorted = pl.pallas_call(
        _gmm_body,
        grid_spec=pltpu.PrefetchScalarGridSpec(
            num_scalar_prefetch=1,
            grid=(nq, h_dim // bh),
            in_specs=[
                pl.BlockSpec((_RBLK, d), lambda q, k, be_r: (q, 0)),
                pl.BlockSpec((1, bh, d), lambda q, k, be_r: (be_r[q], k, 0)),
                pl.BlockSpec((1, bh, d), lambda q, k, be_r: (be_r[q], k, 0)),
                pl.BlockSpec((1, d, bh), lambda q, k, be_r: (be_r[q], 0, k)),
            ],
            out_specs=pl.BlockSpec((_RBLK, d), lambda q, k, be_r: (q, 0)),
        ),
        out_shape=jax.ShapeDtypeStruct((p_max, d), f32),
    )(be, xs_sorted, w_gate, w_up, w_down)

    out = pl.pallas_call(
        _combine_body,
        grid=(n // _RBLK,),
        in_specs=[
            pl.BlockSpec((p_max, d), lambda i: (0, 0)),
            pl.BlockSpec((_RBLK, d), lambda i: (i, 0)),
            pl.BlockSpec(memory_space=pltpu.SMEM),
            pl.BlockSpec(memory_space=pltpu.SMEM),
        ],
        out_specs=pl.BlockSpec((_RBLK, d), lambda i: (i, 0)),
        out_shape=jax.ShapeDtypeStruct((n, d), f32),
    )(eo_sorted, h, dflat, wflat)

    return out.reshape(b, t, d)


# speedup vs baseline: 1.4221x; 1.2337x over previous
import functools

import jax
import jax.numpy as jnp
from jax import lax
from jax.experimental import pallas as pl
from jax.experimental.pallas import tpu as pltpu
from jax.experimental.pallas import tpu_sc as plsc


def _silu(v):
    return v * jax.nn.sigmoid(v)


def _ln(v, w, b):
    m = v.mean(-1, keepdims=True)
    var = ((v - m) ** 2).mean(-1, keepdims=True)
    return (v - m) * jax.lax.rsqrt(var + 1e-5) * w + b


def _inproj_body(x_ref, w_ref, lnw_ref, lnb_ref, o_ref):
    xn = _ln(x_ref[...], lnw_ref[...], lnb_ref[...])
    o_ref[...] = jnp.dot(xn, w_ref[...], preferred_element_type=jnp.float32)


def _conv_body(xin_ref, convw_ref, convb_ref, xpw_ref, dtw_ref, dtb_ref,
               xs_ref, dt_ref, bp_ref, cp_ref, carry_ref):
    nb = pl.program_id(0)

    @pl.when(nb == 0)
    def _():
        carry_ref[...] = jnp.zeros_like(carry_ref)

    xin = xin_ref[...]
    ln = xin.shape[0]
    dc = convw_ref.shape[0]
    ext = jnp.concatenate([carry_ref[...], xin], axis=0)
    acc = jnp.broadcast_to(convb_ref[...], xin.shape)
    for k in range(dc):
        acc = acc + convw_ref[k, :][None, :] * ext[8 + k - (dc - 1): 8 + k - (dc - 1) + ln, :]
    xs = _silu(acc)
    xs_ref[...] = xs
    carry_ref[...] = xin[ln - 8: ln, :]
    xp = jnp.dot(xs, xpw_ref[...], preferred_element_type=jnp.float32)
    r = dtw_ref.shape[0]
    s = bp_ref.shape[1]
    bp_ref[...] = xp[:, r: r + s]
    cp_ref[...] = xp[:, r + s: r + 2 * s]
    dt_ref[...] = jax.nn.softplus(
        jnp.dot(xp[:, :r], dtw_ref[...], preferred_element_type=jnp.float32)
        + dtb_ref[...])


def _scan_body(xs_ref, dt_ref, bp_ref, cp_ref, alogt_ref, dp_ref, y_ref,
               h_ref, da_ref, dbx_ref):
    g = pl.program_id(0)

    @pl.when(g == 0)
    def _():
        h_ref[...] = jnp.zeros_like(h_ref)

    a_neg = -jnp.exp(alogt_ref[...])
    dp = dp_ref[...]
    lc = xs_ref.shape[0]
    u = 64
    outer_dn = (((0,), (0,)), ((), ()))
    contr_dn = (((1,), (0,)), ((), ()))

    def body(k, h):
        j0 = pl.multiple_of(k * u, u)
        dt_c = dt_ref[pl.ds(j0, u), :]
        xs_c = xs_ref[pl.ds(j0, u), :]
        bp_c = bp_ref[pl.ds(j0, u), :]
        cp_c = cp_ref[pl.ds(j0, u), :]
        dtx_c = dt_c * xs_c
        for j in range(u):
            da_ref[j] = jnp.exp(dt_c[j:j + 1, :] * a_neg)
            dbx_ref[j] = jax.lax.dot_general(
                bp_c[j:j + 1, :], dtx_c[j:j + 1, :], outer_dn,
                preferred_element_type=jnp.float32)
        rows = []
        for j in range(u):
            h = da_ref[j] * h + dbx_ref[j]
            rows.append(jax.lax.dot_general(cp_c[j:j + 1, :], h, contr_dn,
                                            preferred_element_type=jnp.float32))
        y_ref[pl.ds(j0, u), :] = jnp.concatenate(rows, axis=0) + dp * xs_c
        return h

    h_ref[...] = jax.lax.fori_loop(0, lc // u, body, h_ref[...])


def _post_body(ys_ref, z_ref, opw_ref, x_ref, ln2w_ref, ln2b_ref, rw_ref,
               h_ref, hn_ref, comb_ref, ints_ref, w12_ref):
    y = ys_ref[...] * _silu(z_ref[...])
    h = x_ref[...] + jnp.dot(y, opw_ref[...], preferred_element_type=jnp.float32)
    h_ref[...] = h
    hn = _ln(h, ln2w_ref[...], ln2b_ref[...])
    hn_ref[...] = hn
    logits = jnp.dot(hn, rw_ref[...], preferred_element_type=jnp.float32)
    p = jax.nn.softmax(logits, axis=-1)
    col = jax.lax.broadcasted_iota(jnp.int32, p.shape, 1)
    i1 = jnp.argmax(p, axis=-1, keepdims=True)
    m1 = jnp.max(p, axis=-1, keepdims=True)
    pm = jnp.where(col == i1, -jnp.inf, p)
    i2 = jnp.argmax(pm, axis=-1, keepdims=True)
    m2 = jnp.max(pm, axis=-1, keepdims=True)
    tot = m1 + m2
    comb_ref[...] = jnp.where(col == i1, m1 / tot,
                              jnp.where(col == i2, m2 / tot, 0.0))
    ints_ref[...] = jnp.concatenate([i1, i2], axis=1).astype(jnp.int32)
    w12_ref[...] = jnp.concatenate([m1 / tot, m2 / tot], axis=1)


_RBLK = 128


def _route_body(comb_ref, ints_ref, dest_ref, w_ref, beq_ref):
    n, e = comb_ref.shape
    nbk = n // _RBLK
    f32 = jnp.float32
    m = (comb_ref[...] != 0.0).astype(f32)
    ri = lax.broadcasted_iota(jnp.int32, (_RBLK, _RBLK), 0)
    ci = lax.broadcasted_iota(jnp.int32, (_RBLK, _RBLK), 1)
    sl = (ci < ri).astype(f32)
    ones_row = jnp.ones((1, _RBLK), f32)
    l16 = (lax.broadcasted_iota(jnp.int32, (nbk, nbk), 1)
           <= lax.broadcasted_iota(jnp.int32, (nbk, nbk), 0)).astype(f32)
    su8 = (lax.broadcasted_iota(jnp.int32, (e, e), 0)
           < lax.broadcasted_iota(jnp.int32, (e, e), 1)).astype(f32)
    dnum = (((1,), (0,)), ((), ()))
    ranks = []
    cnts = []
    for b in range(nbk):
        mb = m[b * _RBLK:(b + 1) * _RBLK, :]
        ranks.append(lax.dot_general(sl, mb, dnum, preferred_element_type=f32))
        cnts.append(lax.dot_general(ones_row, mb, dnum, preferred_element_type=f32))
    pbc = jnp.concatenate(cnts, axis=0)
    p_incl = lax.dot_general(l16, pbc, dnum, preferred_element_type=f32)
    tot = p_incl[nbk - 1:nbk, :]
    ti = tot.astype(jnp.int32)
    ptf = (((ti + (_RBLK - 1)) >> 7) << 7).astype(f32)
    pbase = lax.dot_general(ptf, su8, dnum, preferred_element_type=f32)
    bot = pbase + p_incl - pbc
    col8 = lax.broadcasted_iota(jnp.int32, (_RBLK, e), 1)
    for b in range(nbk):
        dest_b = ranks[b] + bot[b:b + 1, :]
        cb = comb_ref[b * _RBLK:(b + 1) * _RBLK, :]
        i1 = ints_ref[b * _RBLK:(b + 1) * _RBLK, 0:1]
        i2 = ints_ref[b * _RBLK:(b + 1) * _RBLK, 1:2]
        lo = b * _RBLK
        hi = (b + 1) * _RBLK
        for idx, (dlo, dhi) in ((i1, (lo, hi)), (i2, (n + lo, n + hi))):
            sel = col8 == idx
            dest_ref[dlo:dhi, :] = jnp.sum(
                jnp.where(sel, dest_b, 0.0), axis=1, keepdims=True).astype(jnp.int32)
            w_ref[dlo:dhi, :] = jnp.sum(
                jnp.where(sel, cb, 0.0), axis=1, keepdims=True)
    qlo = lax.broadcasted_iota(jnp.int32, beq_ref.shape, 0).astype(f32) * _RBLK
    ecol = lax.broadcasted_iota(jnp.int32, beq_ref.shape, 1).astype(f32)
    inseg = (qlo >= pbase) & (qlo < pbase + ptf)
    be_col = jnp.sum(jnp.where(inseg, ecol, 0.0), axis=1, keepdims=True)
    beq_ref[...] = jnp.broadcast_to(be_col, beq_ref.shape).astype(jnp.int32)


def _gmm_body(be_ref, xr_ref, wg_ref, wu_ref, wd_ref, o_ref):
    xb = xr_ref[...].astype(jnp.bfloat16)
    dn = (((1,), (1,)), ((), ()))
    wg = wg_ref[0].astype(jnp.bfloat16)
    wu = wu_ref[0].astype(jnp.bfloat16)
    wd = wd_ref[0].astype(jnp.bfloat16)
    g = lax.dot_general(xb, wg, dn, preferred_element_type=jnp.float32)
    u = lax.dot_general(xb, wu, dn, preferred_element_type=jnp.float32)
    act = (_silu(g) * u).astype(jnp.bfloat16)
    o_ref[...] = lax.dot_general(act, wd, dn, preferred_element_type=jnp.float32)


def _combine_body(eo_ref, h_ref, d1_ref, d2_ref, w1_ref, w2_ref, o_ref):
    p = eo_ref.shape[0]
    bn = h_ref.shape[0]
    piota = lax.broadcasted_iota(jnp.int32, (bn, p), 1)
    sel = (jnp.where(piota == d1_ref[...], w1_ref[...], 0.0)
           + jnp.where(piota == d2_ref[...], w2_ref[...], 0.0))
    o_ref[...] = h_ref[...] + jnp.dot(sel, eo_ref[...],
                                      preferred_element_type=jnp.float32)


def kernel(x, ln1_w, ln1_b, ln2_w, ln2_b, in_proj_w, conv_w, conv_b, x_proj_w,
           dt_proj_w, dt_proj_b, A_log, Dp, out_proj_w, router_w, w_gate, w_up, w_down):
    b, t, d = x.shape
    di = conv_w.shape[0]
    r = dt_proj_w.shape[1]
    s = A_log.shape[1]
    e = router_w.shape[0]
    h_dim = w_gate.shape[1]
    n = b * t
    f32 = jnp.float32
    xf = x.reshape(n, d)

    ln = 256
    cb = 768
    n_nb = n // ln

    xz = pl.pallas_call(
        _inproj_body,
        grid=(n_nb, (2 * di) // cb),
        in_specs=[
            pl.BlockSpec((ln, d), lambda i, j: (i, 0)),
            pl.BlockSpec((d, cb), lambda i, j: (0, j)),
            pl.BlockSpec((1, d), lambda i, j: (0, 0)),
            pl.BlockSpec((1, d), lambda i, j: (0, 0)),
        ],
        out_specs=pl.BlockSpec((ln, cb), lambda i, j: (i, j)),
        out_shape=jax.ShapeDtypeStruct((n, 2 * di), f32),
    )(xf, in_proj_w.T, ln1_w.reshape(1, d), ln1_b.reshape(1, d))

    xs, dt, bp, cp = pl.pallas_call(
        _conv_body,
        grid=(n_nb,),
        in_specs=[
            pl.BlockSpec((ln, di), lambda i: (i, 0)),
            pl.BlockSpec((conv_w.shape[1], di), lambda i: (0, 0)),
            pl.BlockSpec((1, di), lambda i: (0, 0)),
            pl.BlockSpec((di, r + 2 * s), lambda i: (0, 0)),
            pl.BlockSpec((r, di), lambda i: (0, 0)),
            pl.BlockSpec((1, di), lambda i: (0, 0)),
        ],
        out_specs=[
            pl.BlockSpec((ln, di), lambda i: (i, 0)),
            pl.BlockSpec((ln, di), lambda i: (i, 0)),
            pl.BlockSpec((ln, s), lambda i: (i, 0)),
            pl.BlockSpec((ln, s), lambda i: (i, 0)),
        ],
        out_shape=[
            jax.ShapeDtypeStruct((n, di), f32),
            jax.ShapeDtypeStruct((n, di), f32),
            jax.ShapeDtypeStruct((n, s), f32),
            jax.ShapeDtypeStruct((n, s), f32),
        ],
        scratch_shapes=[pltpu.VMEM((8, di), f32)],
    )(xz, conv_w.T, conv_b.reshape(1, di), x_proj_w.T, dt_proj_w.T,
      dt_proj_b.reshape(1, di))

    lc = 256
    ys = pl.pallas_call(
        _scan_body,
        grid=(n // lc,),
        in_specs=[
            pl.BlockSpec((lc, di), lambda i: (i, 0)),
            pl.BlockSpec((lc, di), lambda i: (i, 0)),
            pl.BlockSpec((lc, s), lambda i: (i, 0)),
            pl.BlockSpec((lc, s), lambda i: (i, 0)),
            pl.BlockSpec((s, di), lambda i: (0, 0)),
            pl.BlockSpec((1, di), lambda i: (0, 0)),
        ],
        out_specs=pl.BlockSpec((lc, di), lambda i: (i, 0)),
        out_shape=jax.ShapeDtypeStruct((n, di), f32),
        scratch_shapes=[pltpu.VMEM((s, di), f32),
                        pltpu.VMEM((64, s, di), f32),
                        pltpu.VMEM((64, s, di), f32)],
    )(xs, dt, bp, cp, A_log.T, Dp.reshape(1, di))

    h, hn, comb, ints12, w12 = pl.pallas_call(
        _post_body,
        grid=(n_nb,),
        in_specs=[
            pl.BlockSpec((ln, di), lambda i: (i, 0)),
            pl.BlockSpec((ln, di), lambda i: (i, 1)),
            pl.BlockSpec((di, d), lambda i: (0, 0)),
            pl.BlockSpec((ln, d), lambda i: (i, 0)),
            pl.BlockSpec((1, d), lambda i: (0, 0)),
            pl.BlockSpec((1, d), lambda i: (0, 0)),
            pl.BlockSpec((d, e), lambda i: (0, 0)),
        ],
        out_specs=[
            pl.BlockSpec((ln, d), lambda i: (i, 0)),
            pl.BlockSpec((ln, d), lambda i: (i, 0)),
            pl.BlockSpec((ln, e), lambda i: (i, 0)),
            pl.BlockSpec((ln, 2), lambda i: (i, 0)),
            pl.BlockSpec((ln, 2), lambda i: (i, 0)),
        ],
        out_shape=[
            jax.ShapeDtypeStruct((n, d), f32),
            jax.ShapeDtypeStruct((n, d), f32),
            jax.ShapeDtypeStruct((n, e), f32),
            jax.ShapeDtypeStruct((n, 2), jnp.int32),
            jax.ShapeDtypeStruct((n, 2), f32),
        ],
    )(ys, xz, out_proj_w.T, xf, ln2_w.reshape(1, d), ln2_b.reshape(1, d),
      router_w.T)

    p_max = 2 * n + e * _RBLK
    nq = p_max // _RBLK
    destf, wf, beq = pl.pallas_call(
        _route_body,
        grid=(1,),
        in_specs=[
            pl.BlockSpec((n, e), lambda i: (0, 0)),
            pl.BlockSpec((n, 2), lambda i: (0, 0)),
        ],
        out_specs=[
            pl.BlockSpec((2 * n, 1), lambda i: (0, 0)),
            pl.BlockSpec((2 * n, 1), lambda i: (0, 0)),
            pl.BlockSpec((_RBLK, e), lambda i: (0, 0)),
        ],
        out_shape=[
            jax.ShapeDtypeStruct((2 * n, 1), jnp.int32),
            jax.ShapeDtypeStruct((2 * n, 1), f32),
            jax.ShapeDtypeStruct((_RBLK, e), jnp.int32),
        ],
    )(comb, ints12)
    be = beq[:nq, 0]
    dflat = destf.reshape(2 * n)
    wflat = wf.reshape(2 * n)

    nw = 32
    mesh = plsc.VectorSubcoreMesh(core_axis_name="c", subcore_axis_name="s")
    a_per_w = (2 * n) // nw

    @functools.partial(
        pl.kernel, mesh=mesh,
        out_type=jax.ShapeDtypeStruct((p_max, d), f32),
        scratch_types=[
            pltpu.VMEM((a_per_w,), jnp.int32),
            pltpu.VMEM((a_per_w, d), f32),
            pltpu.SemaphoreType.DMA,
        ],
    )
    def _sc_disperse(dflat_hbm, hn_hbm, out_hbm, idx_v, rows_v, sem):
        wid = lax.axis_index("s") * 2 + lax.axis_index("c")
        pltpu.sync_copy(dflat_hbm.at[pl.ds(wid * a_per_w, a_per_w)], idx_v)
        tok0 = (wid % 16) * a_per_w
        pltpu.sync_copy(hn_hbm.at[pl.ds(tok0, a_per_w)], rows_v)
        pltpu.async_copy(rows_v, out_hbm.at[idx_v], sem).wait()

    xs_sorted = _sc_disperse(dflat, hn)

    eo_sorted = pl.pallas_call(
        _gmm_body,
        grid_spec=pltpu.PrefetchScalarGridSpec(
            num_scalar_prefetch=1,
            grid=(nq,),
            in_specs=[
                pl.BlockSpec((_RBLK, d), lambda q, be_r: (q, 0)),
                pl.BlockSpec((1, h_dim, d), lambda q, be_r: (be_r[q], 0, 0)),
                pl.BlockSpec((1, h_dim, d), lambda q, be_r: (be_r[q], 0, 0)),
                pl.BlockSpec((1, d, h_dim), lambda q, be_r: (be_r[q], 0, 0)),
            ],
            out_specs=pl.BlockSpec((_RBLK, d), lambda q, be_r: (q, 0)),
        ),
        out_shape=jax.ShapeDtypeStruct((p_max, d), f32),
    )(be, xs_sorted, w_gate, w_up, w_down)

    nb16 = n // _RBLK
    out = pl.pallas_call(
        _combine_body,
        grid=(nb16,),
        in_specs=[
            pl.BlockSpec((p_max, d), lambda i: (0, 0)),
            pl.BlockSpec((_RBLK, d), lambda i: (i, 0)),
            pl.BlockSpec((_RBLK, 1), lambda i: (i, 0)),
            pl.BlockSpec((_RBLK, 1), lambda i, _o=nb16: (i + _o, 0)),
            pl.BlockSpec((_RBLK, 1), lambda i: (i, 0)),
            pl.BlockSpec((_RBLK, 1), lambda i, _o=nb16: (i + _o, 0)),
        ],
        out_specs=pl.BlockSpec((_RBLK, d), lambda i: (i, 0)),
        out_shape=jax.ShapeDtypeStruct((n, d), f32),
    )(eo_sorted, h, destf, destf, wf, wf)

    return out.reshape(b, t, d)
